# baseline (speedup 1.0000x reference)
import functools

import jax
import jax.numpy as jnp
from jax import lax
from jax.experimental import pallas as pl
from jax.experimental.pallas import tpu as pltpu
from jax.experimental.pallas import tpu_sc as plsc

NC = 2
NS = 16
NW = NC * NS

S = 16384
T = 50
D = 64
B = S * T
R = B // 2
R_PER_W = R // NW
CHUNK = 160
N_CHUNKS = R_PER_W // CHUNK
NBUF = 4
MAIN_G = (N_CHUNKS - NBUF) // NBUF
assert R_PER_W % CHUNK == 0 and N_CHUNKS % NBUF == 0 and CHUNK % 8 == 0

_mesh = plsc.VectorSubcoreMesh(core_axis_name="c", subcore_axis_name="s")


@functools.partial(
    pl.kernel,
    out_type=jax.ShapeDtypeStruct((R, 2 * D), jnp.float32),
    mesh=_mesh,
    scratch_types=(
        [pltpu.VMEM((R_PER_W,), jnp.int32) for _ in range(2)]
        + [pltpu.VMEM((CHUNK, D), jnp.float32) for _ in range(2 * NBUF)]
        + [pltpu.SemaphoreType.DMA for _ in range(4 * NBUF)]
    ),
    compiler_params=pltpu.CompilerParams(use_tc_tiling_on_sc=False),
)
def _gather_kernel(ev_hbm, od_hbm, table_hbm, out_hbm, ev_v, od_v, *scratch):
    bufs_l = scratch[:NBUF]
    bufs_r = scratch[NBUF:2 * NBUF]
    gsems_a = scratch[2 * NBUF:3 * NBUF]
    gsems_b = scratch[3 * NBUF:4 * NBUF]
    osems_a = scratch[4 * NBUF:5 * NBUF]
    osems_b = scratch[5 * NBUF:]

    wid = lax.axis_index("s") * NC + lax.axis_index("c")
    wbase = wid * R_PER_W

    def cs(i):
        return pl.ds(pl.multiple_of(i * CHUNK, 8), CHUNK)

    def os_(i):
        return pl.ds(pl.multiple_of(wbase + i * CHUNK, 8), CHUNK)

    hs = pl.ds(pl.multiple_of(wbase, 8), R_PER_W)
    pltpu.sync_copy(ev_hbm.at[hs], ev_v)
    pltpu.sync_copy(od_hbm.at[hs], od_v)

    def fire_gathers(b, i):
        pltpu.async_copy(table_hbm.at[ev_v.at[cs(i)]], bufs_l[b], gsems_a[b])
        pltpu.async_copy(table_hbm.at[od_v.at[cs(i)]], bufs_r[b], gsems_b[b])

    def wait_gathers(b, i):
        pltpu.make_async_copy(table_hbm.at[ev_v.at[cs(i)]], bufs_l[b],
                              gsems_a[b]).wait()
        pltpu.make_async_copy(table_hbm.at[od_v.at[cs(i)]], bufs_r[b],
                              gsems_b[b]).wait()

    def fire_out(b, i):
        pltpu.async_copy(bufs_l[b], out_hbm.at[os_(i), pl.ds(0, D)],
                         osems_a[b])
        pltpu.async_copy(bufs_r[b], out_hbm.at[os_(i), pl.ds(D, D)],
                         osems_b[b])

    def wait_out(b, i):
        pltpu.make_async_copy(bufs_l[b], out_hbm.at[os_(i), pl.ds(0, D)],
                              osems_a[b]).wait()
        pltpu.make_async_copy(bufs_r[b], out_hbm.at[os_(i), pl.ds(D, D)],
                              osems_b[b]).wait()

    for b in range(NBUF):
        fire_gathers(b, b)

    @pl.loop(0, MAIN_G)
    def main(g):
        for b in range(NBUF):
            i = g * NBUF + b
            wait_gathers(b, i)
            fire_out(b, i)
            wait_out(b, i)
            fire_gathers(b, i + NBUF)

    for b in range(NBUF):
        i = MAIN_G * NBUF + b
        wait_gathers(b, i)
        fire_out(b, i)
    for b in range(NBUF):
        i = MAIN_G * NBUF + b
        wait_out(b, i)


_RETILE_SEQS = 32


def _retile_body(x_ref, o_ref):
    x = x_ref[...]
    n = x.shape[0] * 2 // T
    left = x[:, :D].reshape(n, T // 2, 1, D)
    right = x[:, D:].reshape(n, T // 2, 1, D)
    o_ref[...] = jnp.concatenate([left, right], axis=2).reshape(n, T, D)


_retile = pl.pallas_call(
    _retile_body,
    out_shape=jax.ShapeDtypeStruct((S, T, D), jnp.float32),
    grid=(S // _RETILE_SEQS,),
    in_specs=[pl.BlockSpec((_RETILE_SEQS * T // 2, 2 * D), lambda i: (i, 0))],
    out_specs=pl.BlockSpec((_RETILE_SEQS, T, D), lambda i: (i, 0, 0)),
)


def kernel(token_ids, weight):
    flat = token_ids.reshape(-1).astype(jnp.int32)
    pairs = flat.reshape(R, 2)
    out2 = _gather_kernel(pairs[:, 0], pairs[:, 1], weight)
    return _retile(out2)


# speedup vs baseline: 1.4708x; 1.4708x over previous
import functools

import jax
import jax.numpy as jnp
from jax import lax
from jax.experimental import pallas as pl
from jax.experimental.pallas import tpu as pltpu
from jax.experimental.pallas import tpu_sc as plsc

NC = 2
NS = 16
NW = NC * NS

D = 64
B = 16384 * 50
B_PER_W = B // NW
CHUNK = 400
N_CHUNKS = B_PER_W // CHUNK
NBUF = 4
MAIN_G = (N_CHUNKS - NBUF) // NBUF
assert B_PER_W % CHUNK == 0 and N_CHUNKS % NBUF == 0 and CHUNK % 8 == 0

_mesh = plsc.VectorSubcoreMesh(core_axis_name="c", subcore_axis_name="s")


@functools.partial(
    pl.kernel,
    out_type=jax.ShapeDtypeStruct((B, D), jnp.float32),
    mesh=_mesh,
    scratch_types=(
        [pltpu.VMEM((B_PER_W,), jnp.int32)]
        + [pltpu.VMEM((CHUNK, D), jnp.float32) for _ in range(NBUF)]
        + [pltpu.SemaphoreType.DMA for _ in range(2 * NBUF)]
    ),
    compiler_params=pltpu.CompilerParams(use_tc_tiling_on_sc=False),
)
def _gather_kernel(idx_hbm, table_hbm, out_hbm, idx_v, *scratch):
    row_bufs = scratch[:NBUF]
    gsems = scratch[NBUF:2 * NBUF]
    osems = scratch[2 * NBUF:]

    wid = lax.axis_index("s") * NC + lax.axis_index("c")
    wbase = wid * B_PER_W

    def out_slice(i):
        return pl.ds(pl.multiple_of(wbase + i * CHUNK, 8), CHUNK)

    def idx_slice(i):
        return pl.ds(pl.multiple_of(i * CHUNK, 8), CHUNK)

    pltpu.sync_copy(idx_hbm.at[pl.ds(pl.multiple_of(wbase, 8), B_PER_W)],
                    idx_v)

    for b in range(NBUF):
        pltpu.async_copy(
            table_hbm.at[idx_v.at[idx_slice(b)]], row_bufs[b], gsems[b])

    @pl.loop(0, MAIN_G)
    def main(g):
        for b in range(NBUF):
            i = g * NBUF + b
            pltpu.make_async_copy(
                table_hbm.at[idx_v.at[idx_slice(i)]], row_bufs[b],
                gsems[b]).wait()
            pltpu.async_copy(row_bufs[b], out_hbm.at[out_slice(i)], osems[b])
            pltpu.make_async_copy(
                row_bufs[b], out_hbm.at[out_slice(i)], osems[b]).wait()
            pltpu.async_copy(
                table_hbm.at[idx_v.at[idx_slice(i + NBUF)]], row_bufs[b],
                gsems[b])

    for b in range(NBUF):
        i = MAIN_G * NBUF + b
        pltpu.make_async_copy(
            table_hbm.at[idx_v.at[idx_slice(i)]], row_bufs[b], gsems[b]).wait()
        pltpu.async_copy(row_bufs[b], out_hbm.at[out_slice(i)], osems[b])
    for b in range(NBUF):
        i = MAIN_G * NBUF + b
        pltpu.make_async_copy(
            row_bufs[b], out_hbm.at[out_slice(i)], osems[b]).wait()


def kernel(token_ids, weight):
    flat = token_ids.reshape(-1).astype(jnp.int32)
    out = _gather_kernel(flat, weight)
    return out.reshape(token_ids.shape + (weight.shape[1],))
